# baseline (speedup 1.0000x reference)
import jax
import jax.numpy as jnp
from jax import lax
from jax.experimental import pallas as pl
from jax.experimental.pallas import tpu as pltpu
from jax.experimental.pallas import tpu_sc as plsc

DIM = 64
BATCH = 16384

NUM_CORES = 2
NUM_SUBCORES = 16
NUM_WORKERS = NUM_CORES * NUM_SUBCORES
B_PER_W = BATCH // NUM_WORKERS
GCHUNK = 128
NGC = B_PER_W // GCHUNK


def _lookup_body(aid_hbm, pid_hbm, apair_hbm, ppair_hbm, out_hbm,
                 aidx_v, pidx_v, aq_v, pq_v, arows_v, prows_v, out_v,
                 sem0, sem1):
    w = lax.axis_index("s") * NUM_CORES + lax.axis_index("c")
    base = w * B_PER_W
    sems = [sem0, sem1]

    for j in range(NGC):
        pltpu.sync_copy(aid_hbm.at[pl.ds(base + j * GCHUNK, GCHUNK)], aidx_v.at[j])
        pltpu.sync_copy(pid_hbm.at[pl.ds(base + j * GCHUNK, GCHUNK)], pidx_v.at[j])

    for j in range(NGC):
        for g in range(GCHUNK // 16):
            s = pl.ds(g * 16, 16)
            aq_v[j, s] = jnp.right_shift(aidx_v[j, s], 1)
            pq_v[j, s] = jnp.right_shift(pidx_v[j, s], 1)

    def fetch(j, bi):
        pltpu.async_copy(apair_hbm.at[aq_v.at[j]], arows_v.at[bi], sems[bi])
        pltpu.async_copy(ppair_hbm.at[pq_v.at[j]], prows_v.at[bi], sems[bi])

    def drain(j, bi):
        pltpu.make_async_copy(apair_hbm.at[aq_v.at[j]], arows_v.at[bi], sems[bi]).wait()
        pltpu.make_async_copy(ppair_hbm.at[pq_v.at[j]], prows_v.at[bi], sems[bi]).wait()

    lanes = lax.iota(jnp.int32, 16)
    masks = [(lanes & k) != 0 for k in (1, 2, 4, 8)]
    perms = [lanes ^ k for k in (1, 2, 4, 8)]

    def permute(v, idx):
        return v.at[idx].get(mode="promise_in_bounds")

    def merge(x, y, lvl):
        return jnp.where(masks[lvl], y, x) + permute(jnp.where(masks[lvl], x, y), perms[lvl])

    def compute(j, bi):
        for grp in range(GCHUNK // 16):
            s = pl.ds(grp * 16, 16)
            aoff = jnp.bitwise_and(aidx_v[j, s], 1) * DIM
            poff = jnp.bitwise_and(pidx_v[j, s], 1) * DIM
            vs = []
            for rr in range(16):
                r = grp * 16 + rr
                ao = aoff[rr]
                po = poff[rr]
                acc = (arows_v[bi, r, pl.ds(ao, 16)]
                       * prows_v[bi, r, pl.ds(po, 16)])
                for k in range(1, DIM // 16):
                    acc = acc + (arows_v[bi, r, pl.ds(ao + k * 16, 16)]
                                 * prows_v[bi, r, pl.ds(po + k * 16, 16)])
                vs.append(acc)
            for lvl in range(4):
                vs = [merge(vs[2 * i], vs[2 * i + 1], lvl) for i in range(len(vs) // 2)]
            out_v[j, pl.ds(grp * 16, 16)] = vs[0]

    fetch(0, 0)
    fetch(1, 1)
    for j in range(NGC):
        drain(j, j % 2)
        compute(j, j % 2)
        if j + 2 < NGC:
            fetch(j + 2, j % 2)

    for j in range(NGC):
        pltpu.sync_copy(out_v.at[j], out_hbm.at[pl.ds(base + j * GCHUNK, GCHUNK)])


@jax.jit
def _run(author_ids, paper_ids, author_table, paper_table):
    apair = author_table.reshape(author_table.shape[0] // 2, 2 * DIM)
    ppair = paper_table.reshape(paper_table.shape[0] // 2, 2 * DIM)
    mesh = plsc.VectorSubcoreMesh(core_axis_name="c", subcore_axis_name="s")
    return pl.kernel(
        _lookup_body,
        out_type=jax.ShapeDtypeStruct((BATCH,), jnp.float32),
        mesh=mesh,
        scratch_types=[
            pltpu.VMEM((NGC, GCHUNK), jnp.int32),
            pltpu.VMEM((NGC, GCHUNK), jnp.int32),
            pltpu.VMEM((NGC, GCHUNK), jnp.int32),
            pltpu.VMEM((NGC, GCHUNK), jnp.int32),
            pltpu.VMEM((2, GCHUNK, 2 * DIM), jnp.float32),
            pltpu.VMEM((2, GCHUNK, 2 * DIM), jnp.float32),
            pltpu.VMEM((NGC, GCHUNK), jnp.float32),
            pltpu.SemaphoreType.DMA,
            pltpu.SemaphoreType.DMA,
        ],
    )(author_ids, paper_ids, apair, ppair)


def kernel(author_ids, paper_ids, author_table, paper_table):
    return _run(author_ids, paper_ids, author_table, paper_table)


# speedup vs baseline: 1.6597x; 1.6597x over previous
import functools

import jax
import jax.numpy as jnp
from jax import lax
from jax.experimental import pallas as pl
from jax.experimental.pallas import tpu as pltpu
from jax.experimental.pallas import tpu_sc as plsc

DIM = 64
BATCH = 16384

NUM_CORES = 2
NUM_SUBCORES = 16
NUM_WORKERS = NUM_CORES * NUM_SUBCORES
B_PER_W = BATCH // NUM_WORKERS
C = 32
NCHUNK = B_PER_W // C


def _body(aid_hbm, pid_hbm, atab_hbm, ptab_hbm, out_hbm,
          aidx_v, pidx_v, abuf_v, pbuf_v, out_v, *sems):
    wid = lax.axis_index("s") * NUM_CORES + lax.axis_index("c")
    base = wid * B_PER_W

    for j in range(B_PER_W // 128):
        pltpu.sync_copy(aid_hbm.at[pl.ds(base + j * 128, 128)], aidx_v.at[j])
        pltpu.sync_copy(pid_hbm.at[pl.ds(base + j * 128, 128)], pidx_v.at[j])

    asems = sems[:8]
    psems = sems[8:]

    def fetch(g, bi):
        for grp in range(C // 16):
            pos0 = g * C + grp * 16
            avec = aidx_v[pos0 // 128, pl.ds(pos0 % 128, 16)]
            pvec = pidx_v[pos0 // 128, pl.ds(pos0 % 128, 16)]
            for rr in range(16):
                c = grp * 16 + rr
                q = bi * 4 + (c % 4)
                pltpu.async_copy(atab_hbm.at[avec[rr]], abuf_v.at[bi, c], asems[q])
                pltpu.async_copy(ptab_hbm.at[pvec[rr]], pbuf_v.at[bi, c], psems[q])

    def drain(bi):
        for q4 in range(4):
            pltpu.make_async_copy(atab_hbm.at[pl.ds(0, C // 4)],
                                  abuf_v.at[bi, pl.ds(0, C // 4)],
                                  asems[bi * 4 + q4]).wait()
            pltpu.make_async_copy(ptab_hbm.at[pl.ds(0, C // 4)],
                                  pbuf_v.at[bi, pl.ds(0, C // 4)],
                                  psems[bi * 4 + q4]).wait()

    lanes = lax.iota(jnp.int32, 16)
    masks = [(lanes & k) != 0 for k in (1, 2, 4, 8)]
    perms = [lanes ^ k for k in (1, 2, 4, 8)]

    def permute(v, idx):
        return v.at[idx].get(mode="promise_in_bounds")

    def merge(x, y, lvl):
        return jnp.where(masks[lvl], y, x) + permute(jnp.where(masks[lvl], x, y), perms[lvl])

    def compute(g, bi):
        for grp in range(C // 16):
            vs = []
            for rr in range(16):
                c = grp * 16 + rr
                acc = abuf_v[bi, c, pl.ds(0, 16)] * pbuf_v[bi, c, pl.ds(0, 16)]
                for k in range(1, DIM // 16):
                    acc = acc + (abuf_v[bi, c, pl.ds(k * 16, 16)]
                                 * pbuf_v[bi, c, pl.ds(k * 16, 16)])
                vs.append(acc)
            for lvl in range(4):
                vs = [merge(vs[2 * i], vs[2 * i + 1], lvl) for i in range(len(vs) // 2)]
            pos = g * C + grp * 16
            out_v[pos // 128, pl.ds(pos % 128, 16)] = vs[0]

    fetch(0, 0)

    def step(h, _):
        g = h * 2
        fetch(g + 1, 1)
        drain(0)
        compute(g, 0)

        @pl.when(h < NCHUNK // 2 - 1)
        def _():
            fetch(g + 2, 0)

        drain(1)
        compute(g + 1, 1)
        return 0

    lax.fori_loop(0, NCHUNK // 2, step, 0)

    for j in range(B_PER_W // 128):
        pltpu.sync_copy(out_v.at[j], out_hbm.at[pl.ds(base + j * 128, 128)])


@jax.jit
def _run(author_ids, paper_ids, author_table, paper_table):
    mesh = plsc.VectorSubcoreMesh(core_axis_name="c", subcore_axis_name="s")
    return pl.kernel(
        _body,
        out_type=jax.ShapeDtypeStruct((BATCH,), jnp.float32),
        mesh=mesh,
        scratch_types=[
            pltpu.VMEM((B_PER_W // 128, 128), jnp.int32),
            pltpu.VMEM((B_PER_W // 128, 128), jnp.int32),
            pltpu.VMEM((2, C, DIM), jnp.float32),
            pltpu.VMEM((2, C, DIM), jnp.float32),
            pltpu.VMEM((B_PER_W // 128, 128), jnp.float32),
        ] + [pltpu.SemaphoreType.DMA] * 16,
    )(author_ids, paper_ids, author_table, paper_table)


def kernel(author_ids, paper_ids, author_table, paper_table):
    return _run(author_ids, paper_ids, author_table, paper_table)
